# Initial kernel scaffold; baseline (speedup 1.0000x reference)
import jax
import jax.numpy as jnp
from jax.experimental import pallas as pl


def kernel(x, edge_index, batch, gcn_W, gcn_b, fc_W, fc_b, sn_u):
    raise NotImplementedError("write your pallas kernel here")


# speedup vs baseline: 9.9702x; 9.9702x over previous
import functools

import jax
import jax.numpy as jnp
from jax import lax
from jax.experimental import pallas as pl
from jax.experimental.pallas import tpu as pltpu
from jax.experimental.pallas import tpu_sc as plsc

N = 10000
E = 320000
D = 128
OUT = 128
B = 64

NC = 2
NS = 16
NW = NC * NS
EPW = E // NW
CH = 80
NCHUNK = EPW // CH
CAP = EPW + CH
CROWS = CAP // CH + 1

RANGES = (
    (0, 2504, 152, 224),
    (2504, 2504, 152, 224),
    (5008, 2504, 152, 224),
    (7512, 2488, 144, 328),
)
ACC_ROWS = 2560
DUMMY = 2552

RSTEP = 624
RROWS = 640
DEGW = 16


def _sc_degree_body(dst_hbm, out_hbm, dst_all, deg_p):
    cid = lax.axis_index("c")
    sid = lax.axis_index("s")
    w = cid * NS + sid

    pltpu.sync_copy(dst_hbm.at[pl.ds(w * EPW, EPW)], dst_all)

    z16 = jnp.zeros((16,), jnp.float32)
    o16 = jnp.ones((16,), jnp.float32)

    def fill_zeros(i, c):
        deg_p[pl.ds(i * 16, 16)] = z16
        return c

    lax.fori_loop(0, N // 16, fill_zeros, 0)

    def step(i, c):
        dv = dst_all[pl.ds(i * 16, 16)]
        plsc.addupdate_scatter(deg_p, [dv], o16)
        return c

    lax.fori_loop(0, EPW // 16, step, 0)
    pltpu.sync_copy(deg_p, out_hbm.at[w])


def _sc_gather_scatter_body(src_hbm, dst_hbm, y_hbm, out_hbm,
                            src_all, dst_all, comp_src, comp_dst,
                            rows_v, zeros_v, acc_sh, sem):
    cid = lax.axis_index("c")
    sid = lax.axis_index("s")
    w = cid * NS + sid

    pltpu.sync_copy(src_hbm.at[pl.ds(w * EPW, EPW)], src_all)
    pltpu.sync_copy(dst_hbm.at[pl.ds(w * EPW, EPW)], dst_all)

    z16 = jnp.zeros((16,), jnp.float32)
    zmax = max(r[3] for r in RANGES)

    def fill_zeros(i, c):
        for j in range(D // 16):
            zeros_v[i, pl.ds(j * 16, 16)] = z16
        return c

    lax.fori_loop(0, zmax, fill_zeros, 0)

    dummy_s = jnp.zeros((16,), jnp.int32)
    dummy_d = jnp.full((16,), DUMMY, jnp.int32)

    for lo, sz, zstep, zrows in RANGES:
        pltpu.sync_copy(zeros_v.at[pl.ds(0, zrows)],
                        acc_sh.at[pl.ds(sid * zstep, zrows)])
        plsc.subcore_barrier()

        lanes = jnp.arange(16, dtype=jnp.int32)

        def filt(i, off):
            sv = src_all[pl.ds(i * 16, 16)]
            dv = dst_all[pl.ds(i * 16, 16)]
            dl = dv - lo
            m = (dl >= 0) & (dl < sz)
            cs = plsc.cumsum(m.astype(jnp.int32))
            pos = jnp.where(m, off + cs - 1, CAP)
            plsc.store_scatter(comp_src, [pos // CH, pos % CH], sv)
            plsc.store_scatter(comp_dst, [pos // CH, pos % CH], dl)
            return off + cs[15]

        n_p = lax.fori_loop(0, EPW // 16, filt, 0)

        for k in range(CH // 16):
            posk = n_p + k * 16 + lanes
            plsc.store_scatter(comp_src, [posk // CH, posk % CH], dummy_s)
            plsc.store_scatter(comp_dst, [posk // CH, posk % CH], dummy_d)
        nch = (n_p + CH - 1) // CH

        def step(i, c):
            pltpu.async_copy(y_hbm.at[comp_src.at[i]], rows_v, sem).wait()
            pltpu.sync_copy(rows_v, acc_sh.at[comp_dst.at[i]], add=True)
            return c

        lax.fori_loop(0, nch, step, 0)
        plsc.subcore_barrier()
        pltpu.sync_copy(acc_sh.at[pl.ds(sid * zstep, zrows)],
                        out_hbm.at[cid, pl.ds(lo + sid * zstep, zrows)])
        plsc.subcore_barrier()


@functools.cache
def _sc_kernels():
    mesh = plsc.VectorSubcoreMesh(core_axis_name="c", subcore_axis_name="s",
                                  num_cores=NC, num_subcores=NS)
    cparams = pltpu.CompilerParams(needs_layout_passes=False)
    deg_k = pl.kernel(
        _sc_degree_body,
        out_type=jax.ShapeDtypeStruct((NW, N), jnp.float32),
        mesh=mesh,
        compiler_params=cparams,
        scratch_types=[
            pltpu.VMEM((EPW,), jnp.int32),
            pltpu.VMEM((N,), jnp.float32),
        ],
    )
    zmax = max(r[3] for r in RANGES)
    gs_k = pl.kernel(
        _sc_gather_scatter_body,
        out_type=jax.ShapeDtypeStruct((NC, N, D), jnp.float32),
        mesh=mesh,
        compiler_params=cparams,
        scratch_types=[
            pltpu.VMEM((EPW,), jnp.int32),
            pltpu.VMEM((EPW,), jnp.int32),
            pltpu.VMEM((CROWS, CH), jnp.int32),
            pltpu.VMEM((CROWS, CH), jnp.int32),
            pltpu.VMEM((CH, D), jnp.float32),
            pltpu.VMEM((zmax, D), jnp.float32),
            pltpu.VMEM_SHARED((ACC_ROWS, D), jnp.float32),
            pltpu.SemaphoreType.DMA,
        ],
    )
    return deg_k, gs_k


BLK = 1000
NB = N // BLK


def _y_body(x_ref, w_ref, deg_ref, y_ref):
    d = jnp.sum(deg_ref[...], axis=0) + 1.0
    dinv = lax.rsqrt(d)
    xw = lax.dot_general(x_ref[...], w_ref[...], (((1,), (1,)), ((), ())),
                         preferred_element_type=jnp.float32)
    y_ref[...] = xw * dinv


def _pool_body(acc_ref, y_ref, deg_ref, batch_ref, b_ref, out_ref):
    i = pl.program_id(0)

    @pl.when(i == 0)
    def _():
        out_ref[...] = jnp.full((B, D), -jnp.inf, jnp.float32)

    d = jnp.sum(deg_ref[...], axis=0) + 1.0
    dinv = lax.rsqrt(d)
    h = dinv * (acc_ref[0] + acc_ref[1] + y_ref[...]) + b_ref[...]
    h = jnp.where(h >= 0, h, 0.2 * h)

    bcol = batch_ref[...]
    bmin = bcol[0, 0]
    bmax = bcol[BLK - 1, 0]

    def seg(b, c):
        m = bcol == b
        hm = jnp.max(jnp.where(m, h, -jnp.inf), axis=0, keepdims=True)
        cur = out_ref[pl.ds(b, 1), :]
        out_ref[pl.ds(b, 1), :] = jnp.maximum(cur, hm)
        return c

    lax.fori_loop(bmin, bmax + 1, seg, 0)


def _fc_body(pooled_ref, w_ref, u_ref, fcb_ref, out_ref):
    Wm = w_ref[...]
    u = u_ref[...]
    Wv = lax.dot_general(u, Wm, (((1,), (0,)), ((), ())))
    v = Wv / (jnp.sqrt(jnp.sum(Wv * Wv)) + 1e-12)
    uW = lax.dot_general(v, Wm, (((1,), (1,)), ((), ())))
    u_new = uW / (jnp.sqrt(jnp.sum(uW * uW)) + 1e-12)
    sigma = jnp.sum(u_new * uW)
    pw = lax.dot_general(pooled_ref[...], Wm, (((1,), (1,)), ((), ())))
    out_ref[...] = pw / sigma + fcb_ref[...]


def kernel(x, edge_index, batch, gcn_W, gcn_b, fc_W, fc_b, sn_u):
    src = edge_index[0]
    dst = edge_index[1]
    _sc_degree, _sc_gather_scatter = _sc_kernels()

    deg_parts = _sc_degree(dst).reshape(NW, N, 1)

    y = pl.pallas_call(
        _y_body,
        grid=(NB,),
        in_specs=[
            pl.BlockSpec((BLK, D), lambda i: (i, 0)),
            pl.BlockSpec((D, D), lambda i: (0, 0)),
            pl.BlockSpec((NW, BLK, 1), lambda i: (0, i, 0)),
        ],
        out_specs=pl.BlockSpec((BLK, D), lambda i: (i, 0)),
        out_shape=jax.ShapeDtypeStruct((N, D), jnp.float32),
    )(x, gcn_W, deg_parts)

    acc = _sc_gather_scatter(src, dst, y)

    pooled = pl.pallas_call(
        _pool_body,
        grid=(NB,),
        in_specs=[
            pl.BlockSpec((NC, BLK, D), lambda i: (0, i, 0)),
            pl.BlockSpec((BLK, D), lambda i: (i, 0)),
            pl.BlockSpec((NW, BLK, 1), lambda i: (0, i, 0)),
            pl.BlockSpec((BLK, 1), lambda i: (i, 0)),
            pl.BlockSpec((1, D), lambda i: (0, 0)),
        ],
        out_specs=pl.BlockSpec((B, D), lambda i: (0, 0)),
        out_shape=jax.ShapeDtypeStruct((B, D), jnp.float32),
    )(acc, y, deg_parts, batch[:, None], gcn_b[None, :])

    out = pl.pallas_call(
        _fc_body,
        in_specs=[
            pl.BlockSpec((B, D), lambda: (0, 0)),
            pl.BlockSpec((OUT, D), lambda: (0, 0)),
            pl.BlockSpec((1, OUT), lambda: (0, 0)),
            pl.BlockSpec((1, OUT), lambda: (0, 0)),
        ],
        out_specs=pl.BlockSpec((B, OUT), lambda: (0, 0)),
        out_shape=jax.ShapeDtypeStruct((B, OUT), jnp.float32),
    )(pooled, fc_W, sn_u[None, :], fc_b[None, :])

    return out


# speedup vs baseline: 12.6747x; 1.2713x over previous
import functools

import jax
import jax.numpy as jnp
from jax import lax
from jax.experimental import pallas as pl
from jax.experimental.pallas import tpu as pltpu
from jax.experimental.pallas import tpu_sc as plsc

N = 10000
E = 320000
D = 128
OUT = 128
B = 64

NC = 2
NS = 16
NW = NC * NS
EPW = E // NW
CH = 128
NCHUNK = EPW // CH
CDROWS = (EPW + CH - 1) // CH + 1
CAP = CDROWS * CH
CROWS = CDROWS + 1

RANGES = (
    (0, 3336, 200, 336),
    (3336, 3336, 200, 336),
    (6672, 3328, 200, 328),
)
ACC_ROWS = 3344
DUMMY = 3340
ZB = 48


def _sc_degree_body(dst_hbm, out_hbm, dst_all, deg_p):
    cid = lax.axis_index("c")
    sid = lax.axis_index("s")
    w = cid * NS + sid

    pltpu.sync_copy(dst_hbm.at[pl.ds(w * EPW, EPW)], dst_all)

    z16 = jnp.zeros((16,), jnp.float32)
    o16 = jnp.ones((16,), jnp.float32)

    def fill_zeros(i, c):
        deg_p[pl.ds(i * 16, 16)] = z16
        return c

    lax.fori_loop(0, N // 16, fill_zeros, 0)

    def step(i, c):
        dv = dst_all[pl.ds(i * 16, 16)]
        plsc.addupdate_scatter(deg_p, [dv], o16)
        return c

    lax.fori_loop(0, EPW // 16, step, 0)
    pltpu.sync_copy(deg_p, out_hbm.at[w])


def _sc_gather_scatter_body(src_hbm, dst_hbm, y_hbm, out_hbm,
                            src_all, dst_all, comp_src, comp_dst,
                            rows0, rows1, zeros_v, acc_sh, sem0, sem1):
    cid = lax.axis_index("c")
    sid = lax.axis_index("s")
    w = cid * NS + sid

    pltpu.sync_copy(src_hbm.at[pl.ds(w * EPW, EPW)], src_all)
    pltpu.sync_copy(dst_hbm.at[pl.ds(w * EPW, EPW)], dst_all)

    z16 = jnp.zeros((16,), jnp.float32)

    def fill_zeros(i, c):
        for j in range(D // 16):
            zeros_v[i, pl.ds(j * 16, 16)] = z16
        return c

    lax.fori_loop(0, ZB, fill_zeros, 0)

    dummy_s = jnp.zeros((16,), jnp.int32)
    dummy_d = jnp.full((16,), DUMMY, jnp.int32)

    def gather(i, rows, sem):
        return pltpu.make_async_copy(y_hbm.at[comp_src.at[i]], rows, sem)

    for lo, sz, zstep, zrows in RANGES:
        for k in range(336 // ZB):
            pltpu.sync_copy(zeros_v,
                            acc_sh.at[pl.ds(sid * zstep + k * ZB, ZB)])
        plsc.subcore_barrier()

        lanes = jnp.arange(16, dtype=jnp.int32)

        def filt(i, off):
            sv = src_all[pl.ds(i * 16, 16)]
            dv = dst_all[pl.ds(i * 16, 16)]
            dl = dv - lo
            m = (dl >= 0) & (dl < sz)
            cs = plsc.cumsum(m.astype(jnp.int32))
            pos = jnp.where(m, off + cs - 1, CAP)
            plsc.store_scatter(comp_src, [pos // CH, pos % CH], sv)
            plsc.store_scatter(comp_dst, [pos // CH, pos % CH], dl)
            return off + cs[15]

        n_p = lax.fori_loop(0, EPW // 16, filt, 0)

        for k in range(CH // 16):
            posk = n_p + k * 16 + lanes
            plsc.store_scatter(comp_src, [posk // CH, posk % CH], dummy_s)
            plsc.store_scatter(comp_dst, [posk // CH, posk % CH], dummy_d)
        nch = (n_p + CH - 1) // CH

        @pl.when(nch > 0)
        def _():
            gather(0, rows0, sem0).start()

        def pair(j, c):
            i0 = 2 * j
            i1 = i0 + 1

            @pl.when(i0 < nch)
            def _():
                gather(i0, rows0, sem0).wait()

                @pl.when(i1 < nch)
                def _():
                    gather(i1, rows1, sem1).start()

                pltpu.sync_copy(rows0, acc_sh.at[comp_dst.at[i0]], add=True)

            @pl.when(i1 < nch)
            def _():
                gather(i1, rows1, sem1).wait()

                @pl.when(i1 + 1 < nch)
                def _():
                    gather(i1 + 1, rows0, sem0).start()

                pltpu.sync_copy(rows1, acc_sh.at[comp_dst.at[i1]], add=True)

            return c

        lax.fori_loop(0, (nch + 1) // 2, pair, 0)
        plsc.subcore_barrier()
        pltpu.sync_copy(acc_sh.at[pl.ds(sid * zstep, zrows)],
                        out_hbm.at[cid, pl.ds(lo + sid * zstep, zrows)])
        plsc.subcore_barrier()


@functools.cache
def _sc_kernels():
    mesh = plsc.VectorSubcoreMesh(core_axis_name="c", subcore_axis_name="s",
                                  num_cores=NC, num_subcores=NS)
    cparams = pltpu.CompilerParams(needs_layout_passes=False)
    deg_k = pl.kernel(
        _sc_degree_body,
        out_type=jax.ShapeDtypeStruct((NW, N), jnp.float32),
        mesh=mesh,
        compiler_params=cparams,
        scratch_types=[
            pltpu.VMEM((EPW,), jnp.int32),
            pltpu.VMEM((N,), jnp.float32),
        ],
    )
    zmax = max(r[3] for r in RANGES)
    gs_k = pl.kernel(
        _sc_gather_scatter_body,
        out_type=jax.ShapeDtypeStruct((NC, N, D), jnp.float32),
        mesh=mesh,
        compiler_params=cparams,
        scratch_types=[
            pltpu.VMEM((EPW,), jnp.int32),
            pltpu.VMEM((EPW,), jnp.int32),
            pltpu.VMEM((CROWS, CH), jnp.int32),
            pltpu.VMEM((CROWS, CH), jnp.int32),
            pltpu.VMEM((CH, D), jnp.float32),
            pltpu.VMEM((CH, D), jnp.float32),
            pltpu.VMEM((ZB, D), jnp.float32),
            pltpu.VMEM_SHARED((ACC_ROWS, D), jnp.float32),
            pltpu.SemaphoreType.DMA,
            pltpu.SemaphoreType.DMA,
        ],
    )
    return deg_k, gs_k


BLK = 1000
NB = N // BLK


def _y_body(x_ref, w_ref, deg_ref, y_ref):
    d = jnp.sum(deg_ref[...], axis=0) + 1.0
    dinv = lax.rsqrt(d)
    xw = lax.dot_general(x_ref[...], w_ref[...], (((1,), (1,)), ((), ())),
                         preferred_element_type=jnp.float32)
    y_ref[...] = xw * dinv


def _pool_body(acc_ref, y_ref, deg_ref, batch_ref, b_ref, out_ref):
    i = pl.program_id(0)

    @pl.when(i == 0)
    def _():
        out_ref[...] = jnp.full((B, D), -jnp.inf, jnp.float32)

    d = jnp.sum(deg_ref[...], axis=0) + 1.0
    dinv = lax.rsqrt(d)
    h = dinv * (acc_ref[0] + acc_ref[1] + y_ref[...]) + b_ref[...]
    h = jnp.where(h >= 0, h, 0.2 * h)

    bcol = batch_ref[...]
    bmin = bcol[0, 0]
    bmax = bcol[BLK - 1, 0]

    def seg(b, c):
        m = bcol == b
        hm = jnp.max(jnp.where(m, h, -jnp.inf), axis=0, keepdims=True)
        cur = out_ref[pl.ds(b, 1), :]
        out_ref[pl.ds(b, 1), :] = jnp.maximum(cur, hm)
        return c

    lax.fori_loop(bmin, bmax + 1, seg, 0)


def _fc_body(pooled_ref, w_ref, u_ref, fcb_ref, out_ref):
    Wm = w_ref[...]
    u = u_ref[...]
    Wv = lax.dot_general(u, Wm, (((1,), (0,)), ((), ())))
    v = Wv / (jnp.sqrt(jnp.sum(Wv * Wv)) + 1e-12)
    uW = lax.dot_general(v, Wm, (((1,), (1,)), ((), ())))
    u_new = uW / (jnp.sqrt(jnp.sum(uW * uW)) + 1e-12)
    sigma = jnp.sum(u_new * uW)
    pw = lax.dot_general(pooled_ref[...], Wm, (((1,), (1,)), ((), ())))
    out_ref[...] = pw / sigma + fcb_ref[...]


def kernel(x, edge_index, batch, gcn_W, gcn_b, fc_W, fc_b, sn_u):
    src = edge_index[0]
    dst = edge_index[1]
    _sc_degree, _sc_gather_scatter = _sc_kernels()

    deg_parts = _sc_degree(dst).reshape(NW, N, 1)

    y = pl.pallas_call(
        _y_body,
        grid=(NB,),
        in_specs=[
            pl.BlockSpec((BLK, D), lambda i: (i, 0)),
            pl.BlockSpec((D, D), lambda i: (0, 0)),
            pl.BlockSpec((NW, BLK, 1), lambda i: (0, i, 0)),
        ],
        out_specs=pl.BlockSpec((BLK, D), lambda i: (i, 0)),
        out_shape=jax.ShapeDtypeStruct((N, D), jnp.float32),
    )(x, gcn_W, deg_parts)

    acc = _sc_gather_scatter(src, dst, y)

    pooled = pl.pallas_call(
        _pool_body,
        grid=(NB,),
        in_specs=[
            pl.BlockSpec((NC, BLK, D), lambda i: (0, i, 0)),
            pl.BlockSpec((BLK, D), lambda i: (i, 0)),
            pl.BlockSpec((NW, BLK, 1), lambda i: (0, i, 0)),
            pl.BlockSpec((BLK, 1), lambda i: (i, 0)),
            pl.BlockSpec((1, D), lambda i: (0, 0)),
        ],
        out_specs=pl.BlockSpec((B, D), lambda i: (0, 0)),
        out_shape=jax.ShapeDtypeStruct((B, D), jnp.float32),
    )(acc, y, deg_parts, batch[:, None], gcn_b[None, :])

    out = pl.pallas_call(
        _fc_body,
        in_specs=[
            pl.BlockSpec((B, D), lambda: (0, 0)),
            pl.BlockSpec((OUT, D), lambda: (0, 0)),
            pl.BlockSpec((1, OUT), lambda: (0, 0)),
            pl.BlockSpec((1, OUT), lambda: (0, 0)),
        ],
        out_specs=pl.BlockSpec((B, OUT), lambda: (0, 0)),
        out_shape=jax.ShapeDtypeStruct((B, OUT), jnp.float32),
    )(pooled, fc_W, sn_u[None, :], fc_b[None, :])

    return out


# speedup vs baseline: 16.6907x; 1.3168x over previous
import functools

import jax
import jax.numpy as jnp
from jax import lax
from jax.experimental import pallas as pl
from jax.experimental.pallas import tpu as pltpu
from jax.experimental.pallas import tpu_sc as plsc

N = 10000
E = 320000
D = 128
OUT = 128
B = 64

NC = 2
NS = 16
NW = NC * NS
EPW = E // NW
CH = 128
NCHUNK = EPW // CH
CDROWS = (EPW + CH - 1) // CH + 1
CAP = CDROWS * CH
CROWS = CDROWS + 1

RANGES = (
    (0, 3336, 200, 336),
    (3336, 3336, 200, 336),
    (6672, 3328, 200, 328),
)
ACC_ROWS = 3344
DUMMY = 3340
ZB = 48


def _sc_degree_body(dst_hbm, out_hbm, dst_all, deg_p):
    cid = lax.axis_index("c")
    sid = lax.axis_index("s")
    w = cid * NS + sid

    pltpu.sync_copy(dst_hbm.at[pl.ds(w * EPW, EPW)], dst_all)

    z16 = jnp.zeros((16,), jnp.float32)
    o16 = jnp.ones((16,), jnp.float32)

    def fill_zeros(i, c):
        deg_p[pl.ds(i * 16, 16)] = z16
        return c

    lax.fori_loop(0, N // 16, fill_zeros, 0)

    def step(i, c):
        dv = dst_all[pl.ds(i * 16, 16)]
        plsc.addupdate_scatter(deg_p, [dv], o16)
        return c

    lax.fori_loop(0, EPW // 16, step, 0)
    pltpu.sync_copy(deg_p, out_hbm.at[w])


def _sc_gather_scatter_body(src_hbm, dst_hbm, y_hbm, out_hbm,
                            src_all, dst_all, comp_src, comp_dst,
                            rows0, rows1, zeros_v, acc_sh, sem0, sem1):
    cid = lax.axis_index("c")
    sid = lax.axis_index("s")
    w = cid * NS + sid

    pltpu.sync_copy(src_hbm.at[pl.ds(w * EPW, EPW)], src_all)
    pltpu.sync_copy(dst_hbm.at[pl.ds(w * EPW, EPW)], dst_all)

    z16 = jnp.zeros((16,), jnp.float32)

    def fill_zeros(i, c):
        for j in range(D // 16):
            zeros_v[i, pl.ds(j * 16, 16)] = z16
        return c

    lax.fori_loop(0, ZB, fill_zeros, 0)

    dummy_s = jnp.zeros((16,), jnp.int32)
    dummy_d = jnp.full((16,), DUMMY, jnp.int32)

    def gather(i, rows, sem):
        return pltpu.make_async_copy(y_hbm.at[comp_src.at[i]], rows, sem)

    for lo, sz, zstep, zrows in RANGES:
        for k in range(336 // ZB):
            pltpu.sync_copy(zeros_v,
                            acc_sh.at[pl.ds(sid * zstep + k * ZB, ZB)])
        plsc.subcore_barrier()

        lanes = jnp.arange(16, dtype=jnp.int32)

        def filt(i, off):
            sv = src_all[pl.ds(i * 16, 16)]
            dv = dst_all[pl.ds(i * 16, 16)]
            dl = dv - lo
            m = (dl >= 0) & (dl < sz)
            cs = plsc.cumsum(m.astype(jnp.int32))
            pos = jnp.where(m, off + cs - 1, CAP)
            plsc.store_scatter(comp_src, [pos // CH, pos % CH], sv)
            plsc.store_scatter(comp_dst, [pos // CH, pos % CH], dl)
            return off + cs[15]

        n_p = lax.fori_loop(0, EPW // 16, filt, 0)

        for k in range(CH // 16):
            posk = n_p + k * 16 + lanes
            plsc.store_scatter(comp_src, [posk // CH, posk % CH], dummy_s)
            plsc.store_scatter(comp_dst, [posk // CH, posk % CH], dummy_d)
        nch = (n_p + CH - 1) // CH

        @pl.when(nch > 0)
        def _():
            gather(0, rows0, sem0).start()

        def pair(j, c):
            i0 = 2 * j
            i1 = i0 + 1

            @pl.when(i0 < nch)
            def _():
                gather(i0, rows0, sem0).wait()

                @pl.when(i1 < nch)
                def _():
                    gather(i1, rows1, sem1).start()

                pltpu.sync_copy(rows0, acc_sh.at[comp_dst.at[i0]], add=True)

            @pl.when(i1 < nch)
            def _():
                gather(i1, rows1, sem1).wait()

                @pl.when(i1 + 1 < nch)
                def _():
                    gather(i1 + 1, rows0, sem0).start()

                pltpu.sync_copy(rows1, acc_sh.at[comp_dst.at[i1]], add=True)

            return c

        lax.fori_loop(0, (nch + 1) // 2, pair, 0)
        plsc.subcore_barrier()
        pltpu.sync_copy(acc_sh.at[pl.ds(sid * zstep, zrows)],
                        out_hbm.at[cid, pl.ds(lo + sid * zstep, zrows)])
        plsc.subcore_barrier()


@functools.cache
def _sc_kernels():
    mesh = plsc.VectorSubcoreMesh(core_axis_name="c", subcore_axis_name="s",
                                  num_cores=NC, num_subcores=NS)
    cparams = pltpu.CompilerParams(needs_layout_passes=False)
    deg_k = pl.kernel(
        _sc_degree_body,
        out_type=jax.ShapeDtypeStruct((NW, N), jnp.float32),
        mesh=mesh,
        compiler_params=cparams,
        scratch_types=[
            pltpu.VMEM((EPW,), jnp.int32),
            pltpu.VMEM((N,), jnp.float32),
        ],
    )
    zmax = max(r[3] for r in RANGES)
    gs_k = pl.kernel(
        _sc_gather_scatter_body,
        out_type=jax.ShapeDtypeStruct((NC, N, D), jnp.float32),
        mesh=mesh,
        compiler_params=cparams,
        scratch_types=[
            pltpu.VMEM((EPW,), jnp.int32),
            pltpu.VMEM((EPW,), jnp.int32),
            pltpu.VMEM((CROWS, CH), jnp.int32),
            pltpu.VMEM((CROWS, CH), jnp.int32),
            pltpu.VMEM((CH, D), jnp.float32),
            pltpu.VMEM((CH, D), jnp.float32),
            pltpu.VMEM((ZB, D), jnp.float32),
            pltpu.VMEM_SHARED((ACC_ROWS, D), jnp.float32),
            pltpu.SemaphoreType.DMA,
            pltpu.SemaphoreType.DMA,
        ],
    )
    return deg_k, gs_k


BLK = 1000
NB = N // BLK


def _y_body(x_ref, w_ref, deg_ref, y_ref):
    d = deg_ref[:, 0:1] + 1.0
    dinv = lax.rsqrt(d)
    xw = lax.dot_general(x_ref[...], w_ref[...], (((1,), (1,)), ((), ())),
                         preferred_element_type=jnp.float32)
    y_ref[...] = xw * dinv


def _pool_body(acc_ref, y_ref, deg_ref, batch_ref, b_ref, out_ref):
    i = pl.program_id(0)

    @pl.when(i == 0)
    def _():
        out_ref[...] = jnp.full((B, D), -jnp.inf, jnp.float32)

    d = deg_ref[:, 0:1] + 1.0
    dinv = lax.rsqrt(d)
    h = dinv * (acc_ref[0] + acc_ref[1] + y_ref[...]) + b_ref[...]
    h = jnp.where(h >= 0, h, 0.2 * h)

    bcol = batch_ref[...]
    bmin = bcol[0, 0]
    bmax = bcol[BLK - 1, 0]

    def seg(b, c):
        m = bcol == b
        hm = jnp.max(jnp.where(m, h, -jnp.inf), axis=0, keepdims=True)
        cur = out_ref[pl.ds(b, 1), :]
        out_ref[pl.ds(b, 1), :] = jnp.maximum(cur, hm)
        return c

    lax.fori_loop(bmin, bmax + 1, seg, 0)


def _fc_body(pooled_ref, w_ref, u_ref, fcb_ref, out_ref):
    Wm = w_ref[...]
    u = u_ref[...]
    Wv = lax.dot_general(u, Wm, (((1,), (0,)), ((), ())))
    v = Wv / (jnp.sqrt(jnp.sum(Wv * Wv)) + 1e-12)
    uW = lax.dot_general(v, Wm, (((1,), (1,)), ((), ())))
    u_new = uW / (jnp.sqrt(jnp.sum(uW * uW)) + 1e-12)
    sigma = jnp.sum(u_new * uW)
    pw = lax.dot_general(pooled_ref[...], Wm, (((1,), (1,)), ((), ())))
    out_ref[...] = pw / sigma + fcb_ref[...]


def kernel(x, edge_index, batch, gcn_W, gcn_b, fc_W, fc_b, sn_u):
    src = edge_index[0]
    dst = edge_index[1]
    _sc_degree, _sc_gather_scatter = _sc_kernels()

    deg_parts = jnp.broadcast_to(
        jnp.sum(_sc_degree(dst), axis=0)[:, None], (N, 16))

    y = pl.pallas_call(
        _y_body,
        grid=(NB,),
        in_specs=[
            pl.BlockSpec((BLK, D), lambda i: (i, 0)),
            pl.BlockSpec((D, D), lambda i: (0, 0)),
            pl.BlockSpec((BLK, 16), lambda i: (i, 0)),
        ],
        out_specs=pl.BlockSpec((BLK, D), lambda i: (i, 0)),
        out_shape=jax.ShapeDtypeStruct((N, D), jnp.float32),
    )(x, gcn_W, deg_parts)

    acc = _sc_gather_scatter(src, dst, y)

    pooled = pl.pallas_call(
        _pool_body,
        grid=(NB,),
        in_specs=[
            pl.BlockSpec((NC, BLK, D), lambda i: (0, i, 0)),
            pl.BlockSpec((BLK, D), lambda i: (i, 0)),
            pl.BlockSpec((BLK, 16), lambda i: (i, 0)),
            pl.BlockSpec((BLK, 1), lambda i: (i, 0)),
            pl.BlockSpec((1, D), lambda i: (0, 0)),
        ],
        out_specs=pl.BlockSpec((B, D), lambda i: (0, 0)),
        out_shape=jax.ShapeDtypeStruct((B, D), jnp.float32),
    )(acc, y, deg_parts, batch[:, None], gcn_b[None, :])

    out = pl.pallas_call(
        _fc_body,
        in_specs=[
            pl.BlockSpec((B, D), lambda: (0, 0)),
            pl.BlockSpec((OUT, D), lambda: (0, 0)),
            pl.BlockSpec((1, OUT), lambda: (0, 0)),
            pl.BlockSpec((1, OUT), lambda: (0, 0)),
        ],
        out_specs=pl.BlockSpec((B, OUT), lambda: (0, 0)),
        out_shape=jax.ShapeDtypeStruct((B, OUT), jnp.float32),
    )(pooled, fc_W, sn_u[None, :], fc_b[None, :])

    return out
